# baseline (speedup 1.0000x reference)
import functools

import jax
import jax.numpy as jnp
from jax import lax
from jax.experimental import pallas as pl
from jax.experimental.pallas import tpu as pltpu
from jax.experimental.pallas import tpu_sc as plsc

N = 2048
D = 768
E = 64
F = 512
K = 2
T = 64
NTMAX = 128
NPMAX = NTMAX * T
NW = 32

_SC_MESH = plsc.VectorSubcoreMesh(
    core_axis_name="c", subcore_axis_name="s", num_cores=2, num_subcores=16)


def _excl_cumsum_rows(a):
    n = a.shape[0]
    acc = a
    k = 1
    while k < n:
        shifted = jnp.concatenate(
            [jnp.zeros((k,) + a.shape[1:], a.dtype), acc[:-k]], axis=0)
        acc = acc + shifted
        k *= 2
    return acc - a, acc[-1:]


def _cumsum_lanes(a):
    n = a.shape[1]
    acc = a
    k = 1
    while k < n:
        shifted = jnp.concatenate(
            [jnp.zeros((1, k), a.dtype), acc[:, :-k]], axis=1)
        acc = acc + shifted
        k *= 2
    return acc


def _router_body(x_ref, rw_ref, dst_ref, g_ref, cnt_ref):
    x = x_ref[...]
    rw = rw_ref[...]
    logits = lax.dot_general(x, rw, (((1,), (1,)), ((), ())),
                             preferred_element_type=jnp.float32)
    iota = lax.broadcasted_iota(jnp.int32, logits.shape, 1)
    m1 = jnp.max(logits, axis=1, keepdims=True)
    i1 = jnp.min(jnp.where(logits == m1, iota, E), axis=1)
    masked = jnp.where(iota == i1[:, None], -jnp.inf, logits)
    m2 = jnp.max(masked, axis=1, keepdims=True)
    i2 = jnp.min(jnp.where(masked == m2, iota, E), axis=1)
    s = jnp.exp(m2[:, 0] - m1[:, 0])
    g1 = 1.0 / (1.0 + s)
    g2 = s / (1.0 + s)
    oh1 = (iota == i1[:, None]).astype(jnp.int32)
    oh2 = (iota == i2[:, None]).astype(jnp.int32)
    excl, counts = _excl_cumsum_rows(oh1 + oh2)
    cum_nt = _cumsum_lanes((counts + T - 1) // T)
    pad_off = (cum_nt * T) - ((counts + T - 1) // T) * T
    rank1 = jnp.sum(excl * oh1, axis=1)
    rank2 = jnp.sum(excl * oh2, axis=1)
    po1 = jnp.sum(pad_off * oh1, axis=1)
    po2 = jnp.sum(pad_off * oh2, axis=1)
    dst_ref[...] = jnp.stack([po1 + rank1, po2 + rank2], axis=1)
    g_ref[...] = jnp.stack([g1, g2], axis=1)
    cnt_ref[...] = cum_nt


def _router(x2d, router_w):
    return pl.pallas_call(
        _router_body,
        out_shape=(jax.ShapeDtypeStruct((N, K), jnp.int32),
                   jax.ShapeDtypeStruct((N, K), jnp.float32),
                   jax.ShapeDtypeStruct((1, E), jnp.int32)),
    )(x2d, router_w)


@functools.partial(
    pl.kernel,
    out_type=jax.ShapeDtypeStruct((NPMAX, D), jnp.float32),
    mesh=_SC_MESH,
    scratch_types=[
        pltpu.VMEM((K * N // NW,), jnp.int32),
        pltpu.VMEM((K * N // NW,), jnp.int32),
        pltpu.VMEM((K * N // NW, D), jnp.float32),
        pltpu.SemaphoreType.DMA,
        pltpu.SemaphoreType.DMA,
    ],
)
def _dispatch(x_hbm, src_hbm, dst_hbm, xpad_hbm, src_v, dst_v, rows_v,
              sem_g, sem_s):
    per_w = K * N // NW
    wid = lax.axis_index("s") * 2 + lax.axis_index("c")
    base = wid * per_w
    pltpu.sync_copy(src_hbm.at[pl.ds(base, per_w)], src_v)
    pltpu.sync_copy(dst_hbm.at[pl.ds(base, per_w)], dst_v)
    pltpu.async_copy(x_hbm.at[src_v], rows_v, sem_g).wait()
    pltpu.async_copy(rows_v, xpad_hbm.at[dst_v], sem_s).wait()


def _mlp_body(te_ref, rb_ref, x_ref, wi_ref, wo_ref, gp_ref, y_ref):
    del te_ref, rb_ref
    x = x_ref[...].astype(jnp.bfloat16)
    wi = wi_ref[0].astype(jnp.bfloat16)
    h = lax.dot_general(x, wi, (((1,), (1,)), ((), ())),
                        preferred_element_type=jnp.float32)
    g = h[:, :F]
    u = h[:, F:]
    act = (g * jax.nn.sigmoid(g) * u).astype(jnp.bfloat16)
    wo = wo_ref[0].astype(jnp.bfloat16)
    y = lax.dot_general(act, wo, (((1,), (1,)), ((), ())),
                        preferred_element_type=jnp.float32)
    gate = gp_ref[0, 0, :]
    y_ref[...] = y * gate[:, None]


def _mlp(tile_expert, row_block, x_padded, w_in, w_out, gates_pad3):
    grid_spec = pltpu.PrefetchScalarGridSpec(
        num_scalar_prefetch=2,
        grid=(NTMAX,),
        in_specs=[
            pl.BlockSpec((T, D), lambda t, te, rb: (rb[t], 0)),
            pl.BlockSpec((1, 2 * F, D), lambda t, te, rb: (te[t], 0, 0)),
            pl.BlockSpec((1, D, F), lambda t, te, rb: (te[t], 0, 0)),
            pl.BlockSpec((1, 1, T), lambda t, te, rb: (rb[t], 0, 0)),
        ],
        out_specs=pl.BlockSpec((T, D), lambda t, te, rb: (rb[t], 0)),
    )
    return pl.pallas_call(
        _mlp_body,
        grid_spec=grid_spec,
        out_shape=jax.ShapeDtypeStruct((NPMAX, D), jnp.float32),
    )(tile_expert, row_block, x_padded, w_in, w_out, gates_pad3)


@functools.partial(
    pl.kernel,
    out_type=jax.ShapeDtypeStruct((N, D), jnp.float32),
    mesh=_SC_MESH,
    scratch_types=[
        pltpu.VMEM((N // NW,), jnp.int32),
        pltpu.VMEM((N // NW,), jnp.int32),
        pltpu.VMEM((N // NW, D), jnp.float32),
        pltpu.VMEM((N // NW, D), jnp.float32),
        pltpu.SemaphoreType.DMA,
        pltpu.SemaphoreType.DMA,
    ],
)
def _combine(y_hbm, posa_hbm, posb_hbm, out_hbm, ia_v, ib_v, a_v, b_v,
             sem_a, sem_b):
    per_w = N // NW
    wid = lax.axis_index("s") * 2 + lax.axis_index("c")
    base = wid * per_w
    pltpu.sync_copy(posa_hbm.at[pl.ds(base, per_w)], ia_v)
    pltpu.sync_copy(posb_hbm.at[pl.ds(base, per_w)], ib_v)
    ca = pltpu.async_copy(y_hbm.at[ia_v], a_v, sem_a)
    cb = pltpu.async_copy(y_hbm.at[ib_v], b_v, sem_b)
    ca.wait()
    cb.wait()

    def row(r, carry):
        for c in range(D // 16):
            sl = pl.ds(c * 16, 16)
            a_v[r, sl] = a_v[r, sl] + b_v[r, sl]
        return carry

    lax.fori_loop(0, per_w, row, 0)
    pltpu.sync_copy(a_v, out_hbm.at[pl.ds(base, per_w)])


def kernel(layer_input, router_w, w_in, w_out):
    B, S, _ = layer_input.shape
    x2d = layer_input.reshape(N, D)

    dst2, gates, cum_nt = _router(x2d, router_w)

    dst = dst2.reshape(K * N)
    src = jnp.arange(K * N, dtype=jnp.int32) // K
    nt_used = cum_nt[0, E - 1]
    tt = jnp.minimum(jnp.arange(NTMAX, dtype=jnp.int32), nt_used - 1)
    tile_expert = jnp.searchsorted(cum_nt[0], tt, side="right").astype(jnp.int32)
    row_block = tt
    gates_pad = jnp.zeros((NPMAX,), jnp.float32).at[dst].set(gates.reshape(-1))
    posa = dst2[:, 0]
    posb = dst2[:, 1]

    x_padded = _dispatch(x2d, src, dst)
    y_padded = _mlp(tile_expert, row_block, x_padded, w_in, w_out,
                    gates_pad.reshape(NTMAX, 1, T))
    out2d = _combine(y_padded, posa, posb)
    return out2d.reshape(B, S, D)


# speedup vs baseline: 1.3205x; 1.3205x over previous
import functools

import jax
import jax.numpy as jnp
from jax import lax
from jax.experimental import pallas as pl
from jax.experimental.pallas import tpu as pltpu
from jax.experimental.pallas import tpu_sc as plsc

N = 2048
D = 768
E = 64
F = 512
K = 2
T = 64
NTMAX = 128
NPMAX = NTMAX * T
NW = 32

_SC_MESH = plsc.VectorSubcoreMesh(
    core_axis_name="c", subcore_axis_name="s", num_cores=2, num_subcores=16)


def _excl_cumsum_rows(a):
    n = a.shape[0]
    acc = a
    k = 1
    while k < n:
        shifted = jnp.concatenate(
            [jnp.zeros((k,) + a.shape[1:], a.dtype), acc[:-k]], axis=0)
        acc = acc + shifted
        k *= 2
    return acc - a, acc[-1:]


def _cumsum_lanes(a):
    n = a.shape[1]
    acc = a
    k = 1
    while k < n:
        shifted = jnp.concatenate(
            [jnp.zeros((1, k), a.dtype), acc[:, :-k]], axis=1)
        acc = acc + shifted
        k *= 2
    return acc


def _router_body(x_ref, rw_ref, dst_ref, g_ref, cnt_ref):
    x = x_ref[...]
    rw = rw_ref[...]
    logits = lax.dot_general(x, rw, (((1,), (1,)), ((), ())),
                             preferred_element_type=jnp.float32)
    iota = lax.broadcasted_iota(jnp.int32, logits.shape, 1)
    m1 = jnp.max(logits, axis=1, keepdims=True)
    i1 = jnp.min(jnp.where(logits == m1, iota, E), axis=1)
    masked = jnp.where(iota == i1[:, None], -jnp.inf, logits)
    m2 = jnp.max(masked, axis=1, keepdims=True)
    i2 = jnp.min(jnp.where(masked == m2, iota, E), axis=1)
    s = jnp.exp(m2[:, 0] - m1[:, 0])
    g1 = 1.0 / (1.0 + s)
    g2 = s / (1.0 + s)
    oh1 = (iota == i1[:, None]).astype(jnp.int32)
    oh2 = (iota == i2[:, None]).astype(jnp.int32)
    excl, counts = _excl_cumsum_rows(oh1 + oh2)
    cum_nt = _cumsum_lanes((counts + T - 1) // T)
    pad_off = (cum_nt * T) - ((counts + T - 1) // T) * T
    rank1 = jnp.sum(excl * oh1, axis=1)
    rank2 = jnp.sum(excl * oh2, axis=1)
    po1 = jnp.sum(pad_off * oh1, axis=1)
    po2 = jnp.sum(pad_off * oh2, axis=1)
    dst_ref[...] = jnp.stack([po1 + rank1, po2 + rank2], axis=1)
    g_ref[...] = jnp.stack([g1, g2], axis=1)
    cnt_ref[...] = cum_nt


def _router(x2d, router_w):
    return pl.pallas_call(
        _router_body,
        out_shape=(jax.ShapeDtypeStruct((N, K), jnp.int32),
                   jax.ShapeDtypeStruct((N, K), jnp.float32),
                   jax.ShapeDtypeStruct((1, E), jnp.int32)),
    )(x2d, router_w)


@functools.partial(
    pl.kernel,
    out_type=jax.ShapeDtypeStruct((NPMAX, D), jnp.float32),
    mesh=_SC_MESH,
    scratch_types=[
        pltpu.VMEM((K * N // NW,), jnp.int32),
        pltpu.VMEM((K * N // NW,), jnp.int32),
        pltpu.VMEM((K * N // NW, D), jnp.float32),
        pltpu.SemaphoreType.DMA,
        pltpu.SemaphoreType.DMA,
    ],
)
def _dispatch(x_hbm, src_hbm, dst_hbm, xpad_hbm, src_v, dst_v, rows_v,
              sem_g, sem_s):
    per_w = K * N // NW
    wid = lax.axis_index("s") * 2 + lax.axis_index("c")
    base = wid * per_w
    pltpu.sync_copy(src_hbm.at[pl.ds(base, per_w)], src_v)
    pltpu.sync_copy(dst_hbm.at[pl.ds(base, per_w)], dst_v)
    pltpu.async_copy(x_hbm.at[src_v], rows_v, sem_g).wait()
    pltpu.async_copy(rows_v, xpad_hbm.at[dst_v], sem_s).wait()


def _mlp_body(te_ref, rb_ref, kn_ref, fst_ref, runs_ref,
              x_ref, wi_hbm, wo_hbm, gp_ref, y_ref,
              wi_buf, wo_buf, sem_wi, sem_wo):
    t = pl.program_id(0)
    k = kn_ref[t]
    slot = lax.rem(k, 3)

    def fetch(e, s):
        pltpu.make_async_copy(wi_hbm.at[e], wi_buf.at[s], sem_wi.at[s]).start()
        pltpu.make_async_copy(wo_hbm.at[e], wo_buf.at[s], sem_wo.at[s]).start()

    @pl.when(t == 0)
    def _():
        fetch(runs_ref[0], 0)

        @pl.when(runs_ref[1] >= 0)
        def _():
            fetch(runs_ref[1], 1)

    @pl.when(fst_ref[t] == 1)
    def _():
        nxt = runs_ref[k + 2]

        @pl.when(nxt >= 0)
        def _():
            fetch(nxt, lax.rem(k + 2, 3))

        e = te_ref[t]
        pltpu.make_async_copy(wi_hbm.at[e], wi_buf.at[slot],
                              sem_wi.at[slot]).wait()
        pltpu.make_async_copy(wo_hbm.at[e], wo_buf.at[slot],
                              sem_wo.at[slot]).wait()

    @pl.when(rb_ref[t] == t)
    def _():
        x = x_ref[...]
        wi = wi_buf[slot]
        h = lax.dot_general(x, wi, (((1,), (1,)), ((), ())),
                            preferred_element_type=jnp.float32)
        g = h[:, :F]
        u = h[:, F:]
        act = g * jax.nn.sigmoid(g) * u
        wo = wo_buf[slot]
        y = lax.dot_general(act, wo, (((1,), (1,)), ((), ())),
                            preferred_element_type=jnp.float32)
        gate = gp_ref[0, 0, :]
        y_ref[...] = y * gate[:, None]


def _mlp(tile_expert, row_block, knum, isfirst, runs,
         x_padded, w_in, w_out, gates_pad3):
    grid_spec = pltpu.PrefetchScalarGridSpec(
        num_scalar_prefetch=5,
        grid=(NTMAX,),
        in_specs=[
            pl.BlockSpec((T, D), lambda t, te, rb, kn, fs, rn: (rb[t], 0)),
            pl.BlockSpec(memory_space=pl.ANY),
            pl.BlockSpec(memory_space=pl.ANY),
            pl.BlockSpec((1, 1, T), lambda t, te, rb, kn, fs, rn: (rb[t], 0, 0)),
        ],
        out_specs=pl.BlockSpec((T, D), lambda t, te, rb, kn, fs, rn: (rb[t], 0)),
        scratch_shapes=[
            pltpu.VMEM((3, 2 * F, D), jnp.float32),
            pltpu.VMEM((3, D, F), jnp.float32),
            pltpu.SemaphoreType.DMA((3,)),
            pltpu.SemaphoreType.DMA((3,)),
        ],
    )
    return pl.pallas_call(
        _mlp_body,
        grid_spec=grid_spec,
        out_shape=jax.ShapeDtypeStruct((NPMAX, D), jnp.float32),
    )(tile_expert, row_block, knum, isfirst, runs,
      x_padded, w_in, w_out, gates_pad3)


@functools.partial(
    pl.kernel,
    out_type=jax.ShapeDtypeStruct((N, D), jnp.float32),
    mesh=_SC_MESH,
    scratch_types=[
        pltpu.VMEM((N // NW,), jnp.int32),
        pltpu.VMEM((N // NW,), jnp.int32),
        pltpu.VMEM((N // NW, D), jnp.float32),
        pltpu.VMEM((N // NW, D), jnp.float32),
        pltpu.SemaphoreType.DMA,
        pltpu.SemaphoreType.DMA,
    ],
)
def _combine(y_hbm, posa_hbm, posb_hbm, out_hbm, ia_v, ib_v, a_v, b_v,
             sem_a, sem_b):
    per_w = N // NW
    wid = lax.axis_index("s") * 2 + lax.axis_index("c")
    base = wid * per_w
    pltpu.sync_copy(posa_hbm.at[pl.ds(base, per_w)], ia_v)
    pltpu.sync_copy(posb_hbm.at[pl.ds(base, per_w)], ib_v)
    ca = pltpu.async_copy(y_hbm.at[ia_v], a_v, sem_a)
    cb = pltpu.async_copy(y_hbm.at[ib_v], b_v, sem_b)
    ca.wait()
    cb.wait()

    def row(r, carry):
        for c in range(D // 16):
            sl = pl.ds(c * 16, 16)
            a_v[r, sl] = a_v[r, sl] + b_v[r, sl]
        return carry

    lax.fori_loop(0, per_w, row, 0)
    pltpu.sync_copy(a_v, out_hbm.at[pl.ds(base, per_w)])


def kernel(layer_input, router_w, w_in, w_out):
    B, S, _ = layer_input.shape
    x2d = layer_input.reshape(N, D)

    dst2, gates, cum_nt = _router(x2d, router_w)

    dst = dst2.reshape(K * N)
    src = jnp.arange(K * N, dtype=jnp.int32) // K
    nt_used = cum_nt[0, E - 1]
    tt = jnp.minimum(jnp.arange(NTMAX, dtype=jnp.int32), nt_used - 1)
    tile_expert = jnp.searchsorted(cum_nt[0], tt, side="right").astype(jnp.int32)
    row_block = tt
    gates_pad = jnp.zeros((NPMAX,), jnp.float32).at[dst].set(gates.reshape(-1))
    posa = dst2[:, 0]
    posb = dst2[:, 1]

    isfirst = jnp.concatenate(
        [jnp.ones((1,), jnp.int32),
         (tile_expert[1:] != tile_expert[:-1]).astype(jnp.int32)])
    knum = jnp.cumsum(isfirst).astype(jnp.int32) - 1
    runs = jnp.full((NTMAX + 8,), -1, jnp.int32).at[knum].set(tile_expert)

    x_padded = _dispatch(x2d, src, dst)
    y_padded = _mlp(tile_expert, row_block, knum, isfirst, runs,
                    x_padded, w_in, w_out, gates_pad.reshape(NTMAX, 1, T))
    out2d = _combine(y_padded, posa, posb)
    return out2d.reshape(B, S, D)


# speedup vs baseline: 1.3234x; 1.0022x over previous
import functools

import jax
import jax.numpy as jnp
from jax import lax
from jax.experimental import pallas as pl
from jax.experimental.pallas import tpu as pltpu
from jax.experimental.pallas import tpu_sc as plsc

N = 2048
D = 768
E = 64
F = 512
K = 2
T = 64
NTMAX = 128
NPMAX = NTMAX * T
NW = 32

_SC_MESH = plsc.VectorSubcoreMesh(
    core_axis_name="c", subcore_axis_name="s", num_cores=2, num_subcores=16)


def _excl_cumsum_rows(a):
    n = a.shape[0]
    acc = a
    k = 1
    while k < n:
        shifted = jnp.concatenate(
            [jnp.zeros((k,) + a.shape[1:], a.dtype), acc[:-k]], axis=0)
        acc = acc + shifted
        k *= 2
    return acc - a, acc[-1:]


def _cumsum_lanes(a):
    n = a.shape[1]
    acc = a
    k = 1
    while k < n:
        shifted = jnp.concatenate(
            [jnp.zeros((1, k), a.dtype), acc[:, :-k]], axis=1)
        acc = acc + shifted
        k *= 2
    return acc


def _router_body(x_ref, rw_ref, dst_ref, g_ref, cnt_ref):
    x = x_ref[...]
    rw = rw_ref[...]
    logits = lax.dot_general(x, rw, (((1,), (1,)), ((), ())),
                             preferred_element_type=jnp.float32)
    iota = lax.broadcasted_iota(jnp.int32, logits.shape, 1)
    m1 = jnp.max(logits, axis=1, keepdims=True)
    i1 = jnp.min(jnp.where(logits == m1, iota, E), axis=1)
    masked = jnp.where(iota == i1[:, None], -jnp.inf, logits)
    m2 = jnp.max(masked, axis=1, keepdims=True)
    i2 = jnp.min(jnp.where(masked == m2, iota, E), axis=1)
    s = jnp.exp(m2[:, 0] - m1[:, 0])
    g1 = 1.0 / (1.0 + s)
    g2 = s / (1.0 + s)
    oh1 = (iota == i1[:, None]).astype(jnp.int32)
    oh2 = (iota == i2[:, None]).astype(jnp.int32)
    excl, counts = _excl_cumsum_rows(oh1 + oh2)
    cum_nt = _cumsum_lanes((counts + T - 1) // T)
    pad_off = (cum_nt * T) - ((counts + T - 1) // T) * T
    rank1 = jnp.sum(excl * oh1, axis=1)
    rank2 = jnp.sum(excl * oh2, axis=1)
    po1 = jnp.sum(pad_off * oh1, axis=1)
    po2 = jnp.sum(pad_off * oh2, axis=1)
    dst_ref[...] = jnp.stack([po1 + rank1, po2 + rank2], axis=1)
    g_ref[...] = jnp.stack([g1, g2], axis=1)
    cnt_ref[...] = cum_nt


def _router(x2d, router_w):
    return pl.pallas_call(
        _router_body,
        out_shape=(jax.ShapeDtypeStruct((N, K), jnp.int32),
                   jax.ShapeDtypeStruct((N, K), jnp.float32),
                   jax.ShapeDtypeStruct((1, E), jnp.int32)),
    )(x2d, router_w)


@functools.partial(
    pl.kernel,
    out_type=jax.ShapeDtypeStruct((NPMAX, D), jnp.float32),
    mesh=_SC_MESH,
    scratch_types=[
        pltpu.VMEM((K * N // NW,), jnp.int32),
        pltpu.VMEM((K * N // NW,), jnp.int32),
        pltpu.VMEM((K * N // NW, D), jnp.float32),
        pltpu.SemaphoreType.DMA,
        pltpu.SemaphoreType.DMA,
    ],
)
def _dispatch(x_hbm, src_hbm, dst_hbm, xpad_hbm, src_v, dst_v, rows_v,
              sem_g, sem_s):
    per_w = K * N // NW
    wid = lax.axis_index("s") * 2 + lax.axis_index("c")
    base = wid * per_w
    pltpu.sync_copy(src_hbm.at[pl.ds(base, per_w)], src_v)
    pltpu.sync_copy(dst_hbm.at[pl.ds(base, per_w)], dst_v)
    pltpu.async_copy(x_hbm.at[src_v], rows_v, sem_g).wait()
    pltpu.async_copy(rows_v, xpad_hbm.at[dst_v], sem_s).wait()


def _mlp_body(te_ref, rb_ref, kn_ref, fst_ref, runs_ref,
              x_ref, wi_hbm, wo_hbm, gp_ref, y_ref,
              wi_buf, wo_buf, sem_wi, sem_wo):
    t = pl.program_id(0)
    k = kn_ref[t]
    slot = lax.rem(k, 3)

    def fetch(e, s):
        pltpu.make_async_copy(wi_hbm.at[e, pl.ds(0, F)],
                              wi_buf.at[s, pl.ds(0, F)], sem_wi.at[s]).start()
        pltpu.make_async_copy(wi_hbm.at[e, pl.ds(F, F)],
                              wi_buf.at[s, pl.ds(F, F)], sem_wi.at[s]).start()
        pltpu.make_async_copy(wo_hbm.at[e, pl.ds(0, D // 2)],
                              wo_buf.at[s, pl.ds(0, D // 2)],
                              sem_wo.at[s]).start()
        pltpu.make_async_copy(wo_hbm.at[e, pl.ds(D // 2, D // 2)],
                              wo_buf.at[s, pl.ds(D // 2, D // 2)],
                              sem_wo.at[s]).start()

    @pl.when(t == 0)
    def _():
        fetch(runs_ref[0], 0)

        @pl.when(runs_ref[1] >= 0)
        def _():
            fetch(runs_ref[1], 1)

    @pl.when(fst_ref[t] == 1)
    def _():
        nxt = runs_ref[k + 2]

        @pl.when(nxt >= 0)
        def _():
            fetch(nxt, lax.rem(k + 2, 3))

        e = te_ref[t]
        pltpu.make_async_copy(wi_hbm.at[e], wi_buf.at[slot],
                              sem_wi.at[slot]).wait()
        pltpu.make_async_copy(wo_hbm.at[e], wo_buf.at[slot],
                              sem_wo.at[slot]).wait()

    @pl.when(rb_ref[t] == t)
    def _():
        x = x_ref[...]
        wi = wi_buf[slot]
        h = lax.dot_general(x, wi, (((1,), (1,)), ((), ())),
                            preferred_element_type=jnp.float32)
        g = h[:, :F]
        u = h[:, F:]
        act = g * jax.nn.sigmoid(g) * u
        wo = wo_buf[slot]
        y = lax.dot_general(act, wo, (((1,), (1,)), ((), ())),
                            preferred_element_type=jnp.float32)
        gate = gp_ref[0, 0, :]
        y_ref[...] = y * gate[:, None]


def _mlp(tile_expert, row_block, knum, isfirst, runs,
         x_padded, w_in, w_out, gates_pad3):
    grid_spec = pltpu.PrefetchScalarGridSpec(
        num_scalar_prefetch=5,
        grid=(NTMAX,),
        in_specs=[
            pl.BlockSpec((T, D), lambda t, te, rb, kn, fs, rn: (rb[t], 0)),
            pl.BlockSpec(memory_space=pl.ANY),
            pl.BlockSpec(memory_space=pl.ANY),
            pl.BlockSpec((1, 1, T), lambda t, te, rb, kn, fs, rn: (rb[t], 0, 0)),
        ],
        out_specs=pl.BlockSpec((T, D), lambda t, te, rb, kn, fs, rn: (rb[t], 0)),
        scratch_shapes=[
            pltpu.VMEM((3, 2 * F, D), jnp.float32),
            pltpu.VMEM((3, D, F), jnp.float32),
            pltpu.SemaphoreType.DMA((3,)),
            pltpu.SemaphoreType.DMA((3,)),
        ],
    )
    return pl.pallas_call(
        _mlp_body,
        grid_spec=grid_spec,
        out_shape=jax.ShapeDtypeStruct((NPMAX, D), jnp.float32),
    )(tile_expert, row_block, knum, isfirst, runs,
      x_padded, w_in, w_out, gates_pad3)


@functools.partial(
    pl.kernel,
    out_type=jax.ShapeDtypeStruct((N, D), jnp.float32),
    mesh=_SC_MESH,
    scratch_types=[
        pltpu.VMEM((N // NW,), jnp.int32),
        pltpu.VMEM((N // NW,), jnp.int32),
        pltpu.VMEM((N // NW, D), jnp.float32),
        pltpu.VMEM((N // NW, D), jnp.float32),
        pltpu.SemaphoreType.DMA,
        pltpu.SemaphoreType.DMA,
    ],
)
def _combine(y_hbm, posa_hbm, posb_hbm, out_hbm, ia_v, ib_v, a_v, b_v,
             sem_a, sem_b):
    per_w = N // NW
    wid = lax.axis_index("s") * 2 + lax.axis_index("c")
    base = wid * per_w
    pltpu.sync_copy(posa_hbm.at[pl.ds(base, per_w)], ia_v)
    pltpu.sync_copy(posb_hbm.at[pl.ds(base, per_w)], ib_v)
    ca = pltpu.async_copy(y_hbm.at[ia_v], a_v, sem_a)
    cb = pltpu.async_copy(y_hbm.at[ib_v], b_v, sem_b)
    ca.wait()
    cb.wait()

    def row(r, carry):
        for c in range(D // 16):
            sl = pl.ds(c * 16, 16)
            a_v[r, sl] = a_v[r, sl] + b_v[r, sl]
        return carry

    lax.fori_loop(0, per_w, row, 0)
    pltpu.sync_copy(a_v, out_hbm.at[pl.ds(base, per_w)])


def kernel(layer_input, router_w, w_in, w_out):
    B, S, _ = layer_input.shape
    x2d = layer_input.reshape(N, D)

    dst2, gates, cum_nt = _router(x2d, router_w)

    dst = dst2.reshape(K * N)
    src = jnp.arange(K * N, dtype=jnp.int32) // K
    nt_used = cum_nt[0, E - 1]
    tt = jnp.minimum(jnp.arange(NTMAX, dtype=jnp.int32), nt_used - 1)
    tile_expert = jnp.searchsorted(cum_nt[0], tt, side="right").astype(jnp.int32)
    row_block = tt
    gates_pad = jnp.zeros((NPMAX,), jnp.float32).at[dst].set(gates.reshape(-1))
    posa = dst2[:, 0]
    posb = dst2[:, 1]

    isfirst = jnp.concatenate(
        [jnp.ones((1,), jnp.int32),
         (tile_expert[1:] != tile_expert[:-1]).astype(jnp.int32)])
    knum = jnp.cumsum(isfirst).astype(jnp.int32) - 1
    runs = jnp.full((NTMAX + 8,), -1, jnp.int32).at[knum].set(tile_expert)

    x_padded = _dispatch(x2d, src, dst)
    y_padded = _mlp(tile_expert, row_block, knum, isfirst, runs,
                    x_padded, w_in, w_out, gates_pad.reshape(NTMAX, 1, T))
    out2d = _combine(y_padded, posa, posb)
    return out2d.reshape(B, S, D)


# speedup vs baseline: 1.5817x; 1.1953x over previous
import functools

import jax
import jax.numpy as jnp
from jax import lax
from jax.experimental import pallas as pl
from jax.experimental.pallas import tpu as pltpu
from jax.experimental.pallas import tpu_sc as plsc

N = 2048
D = 768
E = 64
F = 512
K = 2
T = 64
NTMAX = 128
NPMAX = NTMAX * T
NW = 32

_SC_MESH = plsc.VectorSubcoreMesh(
    core_axis_name="c", subcore_axis_name="s", num_cores=2, num_subcores=16)


def _excl_cumsum_rows(a):
    n = a.shape[0]
    acc = a
    k = 1
    while k < n:
        shifted = jnp.concatenate(
            [jnp.zeros((k,) + a.shape[1:], a.dtype), acc[:-k]], axis=0)
        acc = acc + shifted
        k *= 2
    return acc - a, acc[-1:]


def _cumsum_lanes(a):
    n = a.shape[1]
    acc = a
    k = 1
    while k < n:
        shifted = jnp.concatenate(
            [jnp.zeros((1, k), a.dtype), acc[:, :-k]], axis=1)
        acc = acc + shifted
        k *= 2
    return acc


def _router_body(x_ref, rw_ref, dst_ref, g_ref):
    x = x_ref[...]
    rw = rw_ref[...]
    logits = lax.dot_general(x, rw, (((1,), (1,)), ((), ())),
                             preferred_element_type=jnp.float32)
    iota = lax.broadcasted_iota(jnp.int32, logits.shape, 1)
    m1 = jnp.max(logits, axis=1, keepdims=True)
    i1 = jnp.min(jnp.where(logits == m1, iota, E), axis=1)
    masked = jnp.where(iota == i1[:, None], -jnp.inf, logits)
    m2 = jnp.max(masked, axis=1, keepdims=True)
    i2 = jnp.min(jnp.where(masked == m2, iota, E), axis=1)
    s = jnp.exp(m2[:, 0] - m1[:, 0])
    g1 = 1.0 / (1.0 + s)
    g2 = s / (1.0 + s)
    oh1 = (iota == i1[:, None]).astype(jnp.int32)
    oh2 = (iota == i2[:, None]).astype(jnp.int32)
    excl, counts = _excl_cumsum_rows(oh1 + oh2)
    cum_nt = _cumsum_lanes((counts + T - 1) // T)
    pad_off = (cum_nt * T) - ((counts + T - 1) // T) * T
    rank1 = jnp.sum(excl * oh1, axis=1)
    rank2 = jnp.sum(excl * oh2, axis=1)
    po1 = jnp.sum(pad_off * oh1, axis=1)
    po2 = jnp.sum(pad_off * oh2, axis=1)
    dst_ref[...] = jnp.stack([po1 + rank1, po2 + rank2], axis=1)
    ones16 = jnp.ones((1, 16), jnp.float32)
    g_ref[...] = jnp.concatenate(
        [g1[:, None] * ones16, g2[:, None] * ones16], axis=1)
    nt = cum_nt[:, E - 1:E]
    tcol = lax.broadcasted_iota(jnp.int32, (NTMAX, 1), 0)
    tcl = jnp.minimum(tcol, nt - 1)
    te = jnp.sum((cum_nt <= tcl).astype(jnp.int32), axis=1, keepdims=True)
    fst = jnp.concatenate(
        [jnp.ones((1, 1), jnp.int32),
         (te[1:] != te[:-1]).astype(jnp.int32)], axis=0)
    kn_excl, _ = _excl_cumsum_rows(fst)
    kn = kn_excl + fst - 1
    used = (counts > 0).astype(jnp.int32)
    cumu = _cumsum_lanes(used)
    m_used = cumu[:, E - 1:E]
    kk = lax.broadcasted_iota(jnp.int32, (NTMAX + 8, 1), 0)
    runs = jnp.sum((cumu <= kk).astype(jnp.int32), axis=1, keepdims=True)
    runs = jnp.where(kk < m_used, runs, -1)
    return te, tcl, fst, kn, runs


def _router_full(x_ref, rw_ref, dst_ref, g_ref, te_ref, rb_ref,
                 fst_ref, kn_ref, runs_ref):
    te, tcl, fst, kn, runs = _router_body(x_ref, rw_ref, dst_ref, g_ref)
    te_ref[...] = te
    rb_ref[...] = tcl
    fst_ref[...] = fst
    kn_ref[...] = kn
    runs_ref[...] = runs


def _router(x2d, router_w):
    return pl.pallas_call(
        _router_full,
        out_shape=(jax.ShapeDtypeStruct((N, K), jnp.int32),
                   jax.ShapeDtypeStruct((N, 2 * 16), jnp.float32),
                   jax.ShapeDtypeStruct((NTMAX, 1), jnp.int32),
                   jax.ShapeDtypeStruct((NTMAX, 1), jnp.int32),
                   jax.ShapeDtypeStruct((NTMAX, 1), jnp.int32),
                   jax.ShapeDtypeStruct((NTMAX, 1), jnp.int32),
                   jax.ShapeDtypeStruct((NTMAX + 8, 1), jnp.int32)),
    )(x2d, router_w)


@functools.partial(
    pl.kernel,
    out_type=jax.ShapeDtypeStruct((NPMAX, D), jnp.float32),
    mesh=_SC_MESH,
    scratch_types=[
        pltpu.VMEM((K * N // NW,), jnp.int32),
        pltpu.VMEM((K * N // NW,), jnp.int32),
        pltpu.VMEM((K * N // NW, D), jnp.float32),
        pltpu.SemaphoreType.DMA,
        pltpu.SemaphoreType.DMA,
    ],
)
def _dispatch(x_hbm, src_hbm, dst_hbm, xpad_hbm, src_v, dst_v, rows_v,
              sem_g, sem_s):
    per_w = K * N // NW
    wid = lax.axis_index("s") * 2 + lax.axis_index("c")
    base = wid * per_w
    pltpu.sync_copy(src_hbm.at[pl.ds(base, per_w)], src_v)
    pltpu.sync_copy(dst_hbm.at[pl.ds(base, per_w)], dst_v)
    pltpu.async_copy(x_hbm.at[src_v], rows_v, sem_g).wait()
    pltpu.async_copy(rows_v, xpad_hbm.at[dst_v], sem_s).wait()


def _mlp_body(te_ref, rb_ref, kn_ref, fst_ref, runs_ref,
              x_ref, wi_hbm, wo_hbm, y_ref,
              wi_buf, wo_buf, sem_wi, sem_wo):
    t = pl.program_id(0)
    k = kn_ref[t]
    slot = lax.rem(k, 3)

    def fetch(e, s):
        pltpu.make_async_copy(wi_hbm.at[e, pl.ds(0, F)],
                              wi_buf.at[s, pl.ds(0, F)], sem_wi.at[s]).start()
        pltpu.make_async_copy(wi_hbm.at[e, pl.ds(F, F)],
                              wi_buf.at[s, pl.ds(F, F)], sem_wi.at[s]).start()
        pltpu.make_async_copy(wo_hbm.at[e, pl.ds(0, D // 2)],
                              wo_buf.at[s, pl.ds(0, D // 2)],
                              sem_wo.at[s]).start()
        pltpu.make_async_copy(wo_hbm.at[e, pl.ds(D // 2, D // 2)],
                              wo_buf.at[s, pl.ds(D // 2, D // 2)],
                              sem_wo.at[s]).start()

    @pl.when(t == 0)
    def _():
        fetch(runs_ref[0], 0)

        @pl.when(runs_ref[1] >= 0)
        def _():
            fetch(runs_ref[1], 1)

    @pl.when(fst_ref[t] == 1)
    def _():
        nxt = runs_ref[k + 2]

        @pl.when(nxt >= 0)
        def _():
            fetch(nxt, lax.rem(k + 2, 3))

        e = te_ref[t]
        pltpu.make_async_copy(wi_hbm.at[e], wi_buf.at[slot],
                              sem_wi.at[slot]).wait()
        pltpu.make_async_copy(wo_hbm.at[e], wo_buf.at[slot],
                              sem_wo.at[slot]).wait()

    @pl.when(rb_ref[t] == t)
    def _():
        rb = rb_ref[t]
        x = x_ref[pl.ds(rb * T, T), :]
        wi = wi_buf[slot]
        h = lax.dot_general(x, wi, (((1,), (1,)), ((), ())),
                            preferred_element_type=jnp.float32)
        g = h[:, :F]
        u = h[:, F:]
        act = g * jax.nn.sigmoid(g) * u
        wo = wo_buf[slot]
        y = lax.dot_general(act, wo, (((1,), (1,)), ((), ())),
                            preferred_element_type=jnp.float32)
        y_ref[...] = y


def _mlp(tile_expert, row_block, knum, isfirst, runs, x_padded, w_in, w_out):
    grid_spec = pltpu.PrefetchScalarGridSpec(
        num_scalar_prefetch=5,
        grid=(NTMAX,),
        in_specs=[
            pl.BlockSpec((NPMAX, D), lambda t, te, rb, kn, fs, rn: (0, 0)),
            pl.BlockSpec(memory_space=pl.ANY),
            pl.BlockSpec(memory_space=pl.ANY),
        ],
        out_specs=pl.BlockSpec((T, D), lambda t, te, rb, kn, fs, rn: (rb[t], 0)),
        scratch_shapes=[
            pltpu.VMEM((3, 2 * F, D), jnp.float32),
            pltpu.VMEM((3, D, F), jnp.float32),
            pltpu.SemaphoreType.DMA((3,)),
            pltpu.SemaphoreType.DMA((3,)),
        ],
    )
    return pl.pallas_call(
        _mlp_body,
        grid_spec=grid_spec,
        out_shape=jax.ShapeDtypeStruct((NPMAX, D), jnp.float32),
    )(tile_expert, row_block, knum, isfirst, runs, x_padded, w_in, w_out)


@functools.partial(
    pl.kernel,
    out_type=jax.ShapeDtypeStruct((N, D), jnp.float32),
    mesh=_SC_MESH,
    scratch_types=[
        pltpu.VMEM((N // NW,), jnp.int32),
        pltpu.VMEM((N // NW,), jnp.int32),
        pltpu.VMEM((N // NW, 2 * 16), jnp.float32),
        pltpu.VMEM((N // NW, D), jnp.float32),
        pltpu.VMEM((N // NW, D), jnp.float32),
        pltpu.SemaphoreType.DMA,
        pltpu.SemaphoreType.DMA,
    ],
)
def _combine(y_hbm, posa_hbm, posb_hbm, gab_hbm, out_hbm,
             ia_v, ib_v, gab_v, a_v, b_v, sem_a, sem_b):
    per_w = N // NW
    wid = lax.axis_index("s") * 2 + lax.axis_index("c")
    base = wid * per_w
    pltpu.sync_copy(posa_hbm.at[pl.ds(base, per_w)], ia_v)
    pltpu.sync_copy(posb_hbm.at[pl.ds(base, per_w)], ib_v)
    pltpu.sync_copy(gab_hbm.at[pl.ds(base, per_w)], gab_v)
    ca = pltpu.async_copy(y_hbm.at[ia_v], a_v, sem_a)
    cb = pltpu.async_copy(y_hbm.at[ib_v], b_v, sem_b)
    ca.wait()
    cb.wait()

    def row(r, carry):
        ga = gab_v[r, pl.ds(0, 16)]
        gb = gab_v[r, pl.ds(16, 16)]
        for c in range(D // 16):
            sl = pl.ds(c * 16, 16)
            a_v[r, sl] = a_v[r, sl] * ga + b_v[r, sl] * gb
        return carry

    lax.fori_loop(0, per_w, row, 0)
    pltpu.sync_copy(a_v, out_hbm.at[pl.ds(base, per_w)])


def kernel(layer_input, router_w, w_in, w_out):
    B, S, _ = layer_input.shape
    x2d = layer_input.reshape(N, D)

    dst2, gab, te, rb, fst, kn, runs = _router(x2d, router_w)

    dst = dst2.reshape(K * N)
    src = jnp.arange(K * N, dtype=jnp.int32) // K
    posa = dst2[:, 0]
    posb = dst2[:, 1]

    x_padded = _dispatch(x2d, src, dst)
    y_padded = _mlp(te.reshape(NTMAX), rb.reshape(NTMAX), kn.reshape(NTMAX),
                    fst.reshape(NTMAX), runs.reshape(NTMAX + 8),
                    x_padded, w_in, w_out)
    out2d = _combine(y_padded, posa, posb, gab)
    return out2d.reshape(B, S, D)


# speedup vs baseline: 1.6084x; 1.0168x over previous
import functools

import jax
import jax.numpy as jnp
from jax import lax
from jax.experimental import pallas as pl
from jax.experimental.pallas import tpu as pltpu
from jax.experimental.pallas import tpu_sc as plsc

N = 2048
D = 768
E = 64
F = 512
K = 2
T = 64
NTMAX = 128
NPMAX = NTMAX * T
NW = 32
NSLOT = 4

_SC_MESH = plsc.VectorSubcoreMesh(
    core_axis_name="c", subcore_axis_name="s", num_cores=2, num_subcores=16)


def _excl_cumsum_rows(a):
    n = a.shape[0]
    acc = a
    k = 1
    while k < n:
        shifted = jnp.concatenate(
            [jnp.zeros((k,) + a.shape[1:], a.dtype), acc[:-k]], axis=0)
        acc = acc + shifted
        k *= 2
    return acc - a, acc[-1:]


def _cumsum_lanes(a):
    n = a.shape[1]
    acc = a
    k = 1
    while k < n:
        shifted = jnp.concatenate(
            [jnp.zeros((1, k), a.dtype), acc[:, :-k]], axis=1)
        acc = acc + shifted
        k *= 2
    return acc


def _router_body(x_ref, rw_ref, dst_ref, g_ref):
    x = x_ref[...]
    rw = rw_ref[...]
    logits = lax.dot_general(x, rw, (((1,), (1,)), ((), ())),
                             preferred_element_type=jnp.float32)
    iota = lax.broadcasted_iota(jnp.int32, logits.shape, 1)
    m1 = jnp.max(logits, axis=1, keepdims=True)
    i1 = jnp.min(jnp.where(logits == m1, iota, E), axis=1)
    masked = jnp.where(iota == i1[:, None], -jnp.inf, logits)
    m2 = jnp.max(masked, axis=1, keepdims=True)
    i2 = jnp.min(jnp.where(masked == m2, iota, E), axis=1)
    s = jnp.exp(m2[:, 0] - m1[:, 0])
    g1 = 1.0 / (1.0 + s)
    g2 = s / (1.0 + s)
    oh1 = (iota == i1[:, None]).astype(jnp.int32)
    oh2 = (iota == i2[:, None]).astype(jnp.int32)
    excl, counts = _excl_cumsum_rows(oh1 + oh2)
    cum_nt = _cumsum_lanes((counts + T - 1) // T)
    pad_off = (cum_nt * T) - ((counts + T - 1) // T) * T
    rank1 = jnp.sum(excl * oh1, axis=1)
    rank2 = jnp.sum(excl * oh2, axis=1)
    po1 = jnp.sum(pad_off * oh1, axis=1)
    po2 = jnp.sum(pad_off * oh2, axis=1)
    dst_ref[...] = jnp.stack([po1 + rank1, po2 + rank2], axis=1)
    ones16 = jnp.ones((1, 16), jnp.float32)
    g_ref[...] = jnp.concatenate(
        [g1[:, None] * ones16, g2[:, None] * ones16], axis=1)
    nt = cum_nt[:, E - 1:E]
    tcol = lax.broadcasted_iota(jnp.int32, (NTMAX, 1), 0)
    tcl = jnp.minimum(tcol, nt - 1)
    te = jnp.sum((cum_nt <= tcl).astype(jnp.int32), axis=1, keepdims=True)
    fst = jnp.concatenate(
        [jnp.ones((1, 1), jnp.int32),
         (te[1:] != te[:-1]).astype(jnp.int32)], axis=0)
    kn_excl, _ = _excl_cumsum_rows(fst)
    kn = kn_excl + fst - 1
    used = (counts > 0).astype(jnp.int32)
    cumu = _cumsum_lanes(used)
    m_used = cumu[:, E - 1:E]
    kk = lax.broadcasted_iota(jnp.int32, (NTMAX + 8, 1), 0)
    runs = jnp.sum((cumu <= kk).astype(jnp.int32), axis=1, keepdims=True)
    runs = jnp.where(kk < m_used, runs, -1)
    return te, tcl, fst, kn, runs


def _router_full(x_ref, rw_ref, dst_ref, g_ref, te_ref, rb_ref,
                 fst_ref, kn_ref, runs_ref):
    te, tcl, fst, kn, runs = _router_body(x_ref, rw_ref, dst_ref, g_ref)
    te_ref[...] = te
    rb_ref[...] = tcl
    fst_ref[...] = fst
    kn_ref[...] = kn
    runs_ref[...] = runs


def _router(x2d, router_w):
    return pl.pallas_call(
        _router_full,
        out_shape=(jax.ShapeDtypeStruct((N, K), jnp.int32),
                   jax.ShapeDtypeStruct((N, 2 * 16), jnp.float32),
                   jax.ShapeDtypeStruct((NTMAX, 1), jnp.int32),
                   jax.ShapeDtypeStruct((NTMAX, 1), jnp.int32),
                   jax.ShapeDtypeStruct((NTMAX, 1), jnp.int32),
                   jax.ShapeDtypeStruct((NTMAX, 1), jnp.int32),
                   jax.ShapeDtypeStruct((NTMAX + 8, 1), jnp.int32)),
    )(x2d, router_w)


@functools.partial(
    pl.kernel,
    out_type=jax.ShapeDtypeStruct((NPMAX, D), jnp.float32),
    mesh=_SC_MESH,
    scratch_types=[
        pltpu.VMEM((K * N // NW,), jnp.int32),
        pltpu.VMEM((K * N // NW,), jnp.int32),
        pltpu.VMEM((K * N // NW, D), jnp.float32),
        pltpu.SemaphoreType.DMA,
        pltpu.SemaphoreType.DMA,
    ],
)
def _dispatch(x_hbm, src_hbm, dst_hbm, xpad_hbm, src_v, dst_v, rows_v,
              sem_g, sem_s):
    per_w = K * N // NW
    wid = lax.axis_index("s") * 2 + lax.axis_index("c")
    base = wid * per_w
    pltpu.sync_copy(src_hbm.at[pl.ds(base, per_w)], src_v)
    pltpu.sync_copy(dst_hbm.at[pl.ds(base, per_w)], dst_v)
    pltpu.async_copy(x_hbm.at[src_v], rows_v, sem_g).wait()
    pltpu.async_copy(rows_v, xpad_hbm.at[dst_v], sem_s).wait()


def _mlp_body(te_ref, rb_ref, kn_ref, fst_ref, runs_ref,
              x_ref, wi_hbm, wo_hbm, y_ref,
              wi_buf, wo_buf, sem_wi, sem_wo):
    t = pl.program_id(0)
    k = kn_ref[t]
    slot = lax.rem(k, NSLOT)

    def fetch(e, s):
        pltpu.make_async_copy(wi_hbm.at[e, pl.ds(0, F)],
                              wi_buf.at[s, pl.ds(0, F)], sem_wi.at[s]).start()
        pltpu.make_async_copy(wi_hbm.at[e, pl.ds(F, F)],
                              wi_buf.at[s, pl.ds(F, F)], sem_wi.at[s]).start()
        pltpu.make_async_copy(wo_hbm.at[e, pl.ds(0, D // 2)],
                              wo_buf.at[s, pl.ds(0, D // 2)],
                              sem_wo.at[s]).start()
        pltpu.make_async_copy(wo_hbm.at[e, pl.ds(D // 2, D // 2)],
                              wo_buf.at[s, pl.ds(D // 2, D // 2)],
                              sem_wo.at[s]).start()

    @pl.when(t == 0)
    def _():
        fetch(runs_ref[0], 0)
        for j in range(1, NSLOT - 1):
            @pl.when(runs_ref[j] >= 0)
            def _(j=j):
                fetch(runs_ref[j], j)

    @pl.when(fst_ref[t] == 1)
    def _():
        nxt = runs_ref[k + (NSLOT - 1)]

        @pl.when(nxt >= 0)
        def _():
            fetch(nxt, lax.rem(k + (NSLOT - 1), NSLOT))

        e = te_ref[t]
        pltpu.make_async_copy(wi_hbm.at[e], wi_buf.at[slot],
                              sem_wi.at[slot]).wait()
        pltpu.make_async_copy(wo_hbm.at[e], wo_buf.at[slot],
                              sem_wo.at[slot]).wait()

    @pl.when(rb_ref[t] == t)
    def _():
        rb = rb_ref[t]
        x = x_ref[pl.ds(rb * T, T), :]
        wi = wi_buf[slot]
        h = lax.dot_general(x, wi, (((1,), (1,)), ((), ())),
                            preferred_element_type=jnp.float32)
        g = h[:, :F]
        u = h[:, F:]
        act = g * jax.nn.sigmoid(g) * u
        wo = wo_buf[slot]
        y = lax.dot_general(act, wo, (((1,), (1,)), ((), ())),
                            preferred_element_type=jnp.float32)
        y_ref[...] = y


def _mlp(tile_expert, row_block, knum, isfirst, runs, x_padded, w_in, w_out):
    grid_spec = pltpu.PrefetchScalarGridSpec(
        num_scalar_prefetch=5,
        grid=(NTMAX,),
        in_specs=[
            pl.BlockSpec((NPMAX, D), lambda t, te, rb, kn, fs, rn: (0, 0)),
            pl.BlockSpec(memory_space=pl.ANY),
            pl.BlockSpec(memory_space=pl.ANY),
        ],
        out_specs=pl.BlockSpec((T, D), lambda t, te, rb, kn, fs, rn: (rb[t], 0)),
        scratch_shapes=[
            pltpu.VMEM((NSLOT, 2 * F, D), jnp.float32),
            pltpu.VMEM((NSLOT, D, F), jnp.float32),
            pltpu.SemaphoreType.DMA((NSLOT,)),
            pltpu.SemaphoreType.DMA((NSLOT,)),
        ],
    )
    return pl.pallas_call(
        _mlp_body,
        grid_spec=grid_spec,
        out_shape=jax.ShapeDtypeStruct((NPMAX, D), jnp.float32),
    )(tile_expert, row_block, knum, isfirst, runs, x_padded, w_in, w_out)


@functools.partial(
    pl.kernel,
    out_type=jax.ShapeDtypeStruct((N, D), jnp.float32),
    mesh=_SC_MESH,
    scratch_types=[
        pltpu.VMEM((N // NW,), jnp.int32),
        pltpu.VMEM((N // NW,), jnp.int32),
        pltpu.VMEM((N // NW, 2 * 16), jnp.float32),
        pltpu.VMEM((N // NW, D), jnp.float32),
        pltpu.VMEM((N // NW, D), jnp.float32),
        pltpu.SemaphoreType.DMA,
        pltpu.SemaphoreType.DMA,
    ],
)
def _combine(y_hbm, posa_hbm, posb_hbm, gab_hbm, out_hbm,
             ia_v, ib_v, gab_v, a_v, b_v, sem_a, sem_b):
    per_w = N // NW
    wid = lax.axis_index("s") * 2 + lax.axis_index("c")
    base = wid * per_w
    pltpu.sync_copy(posa_hbm.at[pl.ds(base, per_w)], ia_v)
    pltpu.sync_copy(posb_hbm.at[pl.ds(base, per_w)], ib_v)
    pltpu.sync_copy(gab_hbm.at[pl.ds(base, per_w)], gab_v)
    ca = pltpu.async_copy(y_hbm.at[ia_v], a_v, sem_a)
    cb = pltpu.async_copy(y_hbm.at[ib_v], b_v, sem_b)
    ca.wait()
    cb.wait()

    def row(r, carry):
        ga = gab_v[r, pl.ds(0, 16)]
        gb = gab_v[r, pl.ds(16, 16)]
        for c in range(D // 16):
            sl = pl.ds(c * 16, 16)
            a_v[r, sl] = a_v[r, sl] * ga + b_v[r, sl] * gb
        return carry

    lax.fori_loop(0, per_w, row, 0)
    pltpu.sync_copy(a_v, out_hbm.at[pl.ds(base, per_w)])


def kernel(layer_input, router_w, w_in, w_out):
    B, S, _ = layer_input.shape
    x2d = layer_input.reshape(N, D)

    dst2, gab, te, rb, fst, kn, runs = _router(x2d, router_w)

    dst = dst2.reshape(K * N)
    src = jnp.arange(K * N, dtype=jnp.int32) // K
    posa = dst2[:, 0]
    posb = dst2[:, 1]

    x_padded = _dispatch(x2d, src, dst)
    y_padded = _mlp(te.reshape(NTMAX), rb.reshape(NTMAX), kn.reshape(NTMAX),
                    fst.reshape(NTMAX), runs.reshape(NTMAX + 8),
                    x_padded, w_in, w_out)
    out2d = _combine(y_padded, posa, posb, gab)
    return out2d.reshape(B, S, D)
